# baseline (speedup 1.0000x reference)
import functools
import jax
import jax.numpy as jnp
from jax import lax
from jax.experimental import pallas as pl
from jax.experimental.pallas import tpu as pltpu
from jax.experimental.pallas import tpu_sc as plsc

NUM_WORKERS = 32
NBUF = 2
WIDE = 2
PAIR = 2


def _make_sc_kernel(b, lw, d):
    bpw = b // NUM_WORKERS
    ppw = bpw // PAIR
    mesh = plsc.VectorSubcoreMesh(core_axis_name="c", subcore_axis_name="s")

    @functools.partial(
        pl.kernel,
        mesh=mesh,
        out_type=jax.ShapeDtypeStruct((b * lw, WIDE, d), jnp.float32),
        scratch_types=[
            pltpu.VMEM((ppw, 104), jnp.int32),
            pltpu.VMEM((ppw, 96), jnp.int32),
        ]
        + [pltpu.VMEM((PAIR * lw, WIDE, d), jnp.float32) for _ in range(NBUF)]
        + [pltpu.SemaphoreType.DMA] * (3 * NBUF),
    )
    def sc_kernel(x_hbm, pos_hbm, idx_a_hbm, idx_b_hbm, out_hbm, idx_a_v,
                  idx_b_v, *rest):
        bufs = rest[:NBUF]
        s_init = rest[NBUF:2 * NBUF]
        s_gadd = rest[2 * NBUF:3 * NBUF]
        s_out = rest[3 * NBUF:4 * NBUF]
        cid = lax.axis_index("c")
        sid = lax.axis_index("s")
        wid = sid * 2 + cid

        pltpu.sync_copy(idx_a_hbm.at[pl.ds(wid * ppw, ppw)], idx_a_v)
        pltpu.sync_copy(idx_b_hbm.at[pl.ds(wid * ppw, ppw)], idx_b_v)

        init_h = [None] * ppw
        gadd_h = [None] * ppw
        scat_h = [None] * ppw

        def issue_init(j):
            init_h[j] = pltpu.async_copy(pos_hbm, bufs[j % NBUF],
                                         s_init[j % NBUF])

        def issue_scat(j):
            scat_h[j] = pltpu.async_copy(
                bufs[j % NBUF],
                out_hbm.at[pl.ds((wid * bpw + j * PAIR) * lw, PAIR * lw)],
                s_out[j % NBUF])

        issue_init(0)
        for i in range(ppw):
            p = i % NBUF
            if i >= 1:
                gadd_h[i - 1][0].wait()
                gadd_h[i - 1][1].wait()
                issue_scat(i - 1)
            init_h[i].wait()
            gadd_h[i] = (
                pltpu.async_copy(
                    x_hbm.at[idx_a_v.at[i]],
                    bufs[p].at[pl.ds(0, 104)], s_gadd[p], add=True),
                pltpu.async_copy(
                    x_hbm.at[idx_b_v.at[i]],
                    bufs[p].at[pl.ds(104, 96)], s_gadd[p], add=True),
            )
            if i + 1 < ppw:
                if i >= 1:
                    scat_h[i - 1].wait()
                issue_init(i + 1)
        gadd_h[ppw - 1][0].wait()
        gadd_h[ppw - 1][1].wait()
        issue_scat(ppw - 1)
        scat_h[ppw - 2].wait()
        scat_h[ppw - 1].wait()

    return sc_kernel


def kernel(x, pos_table):
    b, l, d = x.shape
    lw = l // WIDE
    dw = d * WIDE
    x2 = x.reshape(b * lw, WIDE, d)
    pos2 = jnp.tile(pos_table.reshape(lw, WIDE, d), (PAIR, 1, 1))
    rows = jnp.arange(b * lw, dtype=jnp.int32).reshape(b // PAIR, PAIR * lw)
    idx_a = rows[:, :104]
    idx_b = rows[:, 104:]
    out = _make_sc_kernel(b, lw, d)(x2, pos2, idx_a, idx_b)
    return out.reshape(b, l, d)


# speedup vs baseline: 1.9054x; 1.9054x over previous
import functools
import jax
import jax.numpy as jnp
from jax import lax
from jax.experimental import pallas as pl
from jax.experimental.pallas import tpu as pltpu
from jax.experimental.pallas import tpu_sc as plsc

NUM_WORKERS = 32
NBUF = 4
WIDE = 2
PAIR = 2
HALF_A = 104
HALF_B = 96


def _make_sc_kernel(b, lw, d):
    bpw = b // NUM_WORKERS
    ppw = bpw // PAIR
    nst = 2 * ppw
    mesh = plsc.VectorSubcoreMesh(core_axis_name="c", subcore_axis_name="s")

    @functools.partial(
        pl.kernel,
        mesh=mesh,
        out_type=jax.ShapeDtypeStruct((b * lw, WIDE, d), jnp.float32),
        scratch_types=[
            pltpu.VMEM((ppw, HALF_A), jnp.int32),
            pltpu.VMEM((ppw, HALF_B), jnp.int32),
            pltpu.VMEM_SHARED((PAIR * lw, WIDE, d), jnp.float32),
        ]
        + [pltpu.VMEM((HALF_A, WIDE, d), jnp.float32) for _ in range(NBUF)]
        + [pltpu.SemaphoreType.DMA] * (3 * NBUF),
    )
    def sc_kernel(x_hbm, pos_hbm, idx_a_hbm, idx_b_hbm, out_hbm, idx_a_v,
                  idx_b_v, pos_sh, *rest):
        bufs = rest[:NBUF]
        s_init = rest[NBUF:2 * NBUF]
        s_gadd = rest[2 * NBUF:3 * NBUF]
        s_out = rest[3 * NBUF:4 * NBUF]
        cid = lax.axis_index("c")
        sid = lax.axis_index("s")
        wid = sid * 2 + cid

        @pl.when(sid == 0)
        def _():
            pltpu.sync_copy(pos_hbm, pos_sh)

        pltpu.sync_copy(idx_a_hbm.at[pl.ds(wid * ppw, ppw)], idx_a_v)
        pltpu.sync_copy(idx_b_hbm.at[pl.ds(wid * ppw, ppw)], idx_b_v)
        plsc.subcore_barrier()

        init_h = [None] * nst
        gadd_h = [None] * nst
        scat_h = [None] * nst

        def step_geom(t):
            k, h = divmod(t, 2)
            off = 0 if h == 0 else HALF_A
            n = HALF_A if h == 0 else HALF_B
            return k, off, n

        def issue_init(t):
            _, off, n = step_geom(t)
            init_h[t] = pltpu.async_copy(
                pos_sh.at[pl.ds(off, n)], bufs[t % NBUF].at[pl.ds(0, n)],
                s_init[t % NBUF])

        def issue_gadd(t):
            k, _, n = step_geom(t)
            idx = idx_a_v if t % 2 == 0 else idx_b_v
            gadd_h[t] = pltpu.async_copy(
                x_hbm.at[idx.at[k]], bufs[t % NBUF].at[pl.ds(0, n)],
                s_gadd[t % NBUF], add=True)

        def issue_scat(t):
            k, off, n = step_geom(t)
            row0 = (wid * bpw + k * PAIR) * lw + off
            scat_h[t] = pltpu.async_copy(
                bufs[t % NBUF].at[pl.ds(0, n)],
                out_hbm.at[pl.ds(row0, n)], s_out[t % NBUF])

        issue_init(0)
        for t in range(nst):
            if t >= 1:
                gadd_h[t - 1].wait()
                issue_scat(t - 1)
            init_h[t].wait()
            issue_gadd(t)
            if t + 1 < nst:
                if t >= NBUF - 1:
                    scat_h[t - (NBUF - 1)].wait()
                issue_init(t + 1)
        gadd_h[nst - 1].wait()
        issue_scat(nst - 1)
        for j in range(nst - NBUF + 1, nst):
            scat_h[j].wait()

    return sc_kernel


def kernel(x, pos_table):
    b, l, d = x.shape
    lw = l // WIDE
    x2 = x.reshape(b * lw, WIDE, d)
    pos2 = jnp.tile(pos_table.reshape(lw, WIDE, d), (PAIR, 1, 1))
    rows = jnp.arange(b * lw, dtype=jnp.int32).reshape(b // PAIR, PAIR * lw)
    idx_a = rows[:, :HALF_A]
    idx_b = rows[:, HALF_A:]
    out = _make_sc_kernel(b, lw, d)(x2, pos2, idx_a, idx_b)
    return out.reshape(b, l, d)


# speedup vs baseline: 1.9443x; 1.0204x over previous
import functools
import jax
import jax.numpy as jnp
from jax import lax
from jax.experimental import pallas as pl
from jax.experimental.pallas import tpu as pltpu
from jax.experimental.pallas import tpu_sc as plsc

NUM_WORKERS = 32
NBUF = 2
WIDE = 2
PAIR = 2


def _make_sc_kernel(b, lw, d):
    bpw = b // NUM_WORKERS
    ppw = bpw // PAIR
    mesh = plsc.VectorSubcoreMesh(core_axis_name="c", subcore_axis_name="s")

    @functools.partial(
        pl.kernel,
        mesh=mesh,
        out_type=jax.ShapeDtypeStruct((b * lw, WIDE, d), jnp.float32),
        scratch_types=[
            pltpu.VMEM((ppw, 104), jnp.int32),
            pltpu.VMEM((ppw, 96), jnp.int32),
            pltpu.VMEM_SHARED((PAIR * lw, WIDE, d), jnp.float32),
        ]
        + [pltpu.VMEM((PAIR * lw, WIDE, d), jnp.float32) for _ in range(NBUF)]
        + [pltpu.SemaphoreType.DMA] * (3 * NBUF),
    )
    def sc_kernel(x_hbm, pos_hbm, idx_a_hbm, idx_b_hbm, out_hbm, idx_a_v,
                  idx_b_v, pos_sh, *rest):
        bufs = rest[:NBUF]
        s_init = rest[NBUF:2 * NBUF]
        s_gadd = rest[2 * NBUF:3 * NBUF]
        s_out = rest[3 * NBUF:4 * NBUF]
        cid = lax.axis_index("c")
        sid = lax.axis_index("s")
        wid = sid * 2 + cid

        @pl.when(sid == 0)
        def _():
            pltpu.sync_copy(pos_hbm, pos_sh)

        pltpu.sync_copy(idx_a_hbm.at[pl.ds(wid * ppw, ppw)], idx_a_v)
        pltpu.sync_copy(idx_b_hbm.at[pl.ds(wid * ppw, ppw)], idx_b_v)
        plsc.subcore_barrier()

        init_h = [None] * ppw
        gadd_h = [None] * ppw
        scat_h = [None] * ppw

        def issue_init(j):
            init_h[j] = pltpu.async_copy(pos_sh, bufs[j % NBUF],
                                         s_init[j % NBUF])

        def issue_scat(j):
            scat_h[j] = pltpu.async_copy(
                bufs[j % NBUF],
                out_hbm.at[pl.ds((wid * bpw + j * PAIR) * lw, PAIR * lw)],
                s_out[j % NBUF])

        issue_init(0)
        for i in range(ppw):
            p = i % NBUF
            if i >= 1:
                gadd_h[i - 1][0].wait()
                gadd_h[i - 1][1].wait()
                issue_scat(i - 1)
            init_h[i].wait()
            gadd_h[i] = (
                pltpu.async_copy(
                    x_hbm.at[idx_a_v.at[i]],
                    bufs[p].at[pl.ds(0, 104)], s_gadd[p], add=True),
                pltpu.async_copy(
                    x_hbm.at[idx_b_v.at[i]],
                    bufs[p].at[pl.ds(104, 96)], s_gadd[p], add=True),
            )
            if i + 1 < ppw:
                if i >= 1:
                    scat_h[i - 1].wait()
                issue_init(i + 1)
        gadd_h[ppw - 1][0].wait()
        gadd_h[ppw - 1][1].wait()
        issue_scat(ppw - 1)
        scat_h[ppw - 2].wait()
        scat_h[ppw - 1].wait()

    return sc_kernel


def kernel(x, pos_table):
    b, l, d = x.shape
    lw = l // WIDE
    dw = d * WIDE
    x2 = x.reshape(b * lw, WIDE, d)
    pos2 = jnp.tile(pos_table.reshape(lw, WIDE, d), (PAIR, 1, 1))
    rows = jnp.arange(b * lw, dtype=jnp.int32).reshape(b // PAIR, PAIR * lw)
    idx_a = rows[:, :104]
    idx_b = rows[:, 104:]
    out = _make_sc_kernel(b, lw, d)(x2, pos2, idx_a, idx_b)
    return out.reshape(b, l, d)


# speedup vs baseline: 1.9547x; 1.0054x over previous
import functools
import jax
import jax.numpy as jnp
from jax import lax
from jax.experimental import pallas as pl
from jax.experimental.pallas import tpu as pltpu
from jax.experimental.pallas import tpu_sc as plsc

NUM_WORKERS = 32
NBUF = 2
WIDE = 2
PAIR = 2


def _make_sc_kernel(b, lw, d):
    bpw = b // NUM_WORKERS
    ppw = bpw // PAIR
    mesh = plsc.VectorSubcoreMesh(core_axis_name="c", subcore_axis_name="s")

    @functools.partial(
        pl.kernel,
        mesh=mesh,
        out_type=jax.ShapeDtypeStruct((b * lw, WIDE, d), jnp.float32),
        scratch_types=[
            pltpu.VMEM((ppw, 128), jnp.int32),
            pltpu.VMEM((ppw, 72), jnp.int32),
            pltpu.VMEM_SHARED((PAIR * lw, WIDE, d), jnp.float32),
        ]
        + [pltpu.VMEM((PAIR * lw, WIDE, d), jnp.float32) for _ in range(NBUF)]
        + [pltpu.SemaphoreType.DMA] * (3 * NBUF),
    )
    def sc_kernel(x_hbm, pos_hbm, idx_a_hbm, idx_b_hbm, out_hbm, idx_a_v,
                  idx_b_v, pos_sh, *rest):
        bufs = rest[:NBUF]
        s_init = rest[NBUF:2 * NBUF]
        s_gadd = rest[2 * NBUF:3 * NBUF]
        s_out = rest[3 * NBUF:4 * NBUF]
        cid = lax.axis_index("c")
        sid = lax.axis_index("s")
        wid = sid * 2 + cid

        @pl.when(sid == 0)
        def _():
            pltpu.sync_copy(pos_hbm, pos_sh)

        pltpu.sync_copy(idx_a_hbm.at[pl.ds(wid * ppw, ppw)], idx_a_v)
        pltpu.sync_copy(idx_b_hbm.at[pl.ds(wid * ppw, ppw)], idx_b_v)
        plsc.subcore_barrier()

        init_h = [None] * ppw
        gadd_h = [None] * ppw
        scat_h = [None] * ppw

        def issue_init(j):
            init_h[j] = pltpu.async_copy(pos_sh, bufs[j % NBUF],
                                         s_init[j % NBUF])

        def issue_scat(j):
            scat_h[j] = pltpu.async_copy(
                bufs[j % NBUF],
                out_hbm.at[pl.ds((wid * bpw + j * PAIR) * lw, PAIR * lw)],
                s_out[j % NBUF])

        issue_init(0)
        for i in range(ppw):
            p = i % NBUF
            if i >= 1:
                gadd_h[i - 1][0].wait()
                gadd_h[i - 1][1].wait()
                issue_scat(i - 1)
            init_h[i].wait()
            gadd_h[i] = (
                pltpu.async_copy(
                    x_hbm.at[idx_a_v.at[i]],
                    bufs[p].at[pl.ds(0, 128)], s_gadd[p], add=True),
                pltpu.async_copy(
                    x_hbm.at[idx_b_v.at[i]],
                    bufs[p].at[pl.ds(128, 72)], s_gadd[p], add=True),
            )
            if i + 1 < ppw:
                if i >= 1:
                    scat_h[i - 1].wait()
                issue_init(i + 1)
        gadd_h[ppw - 1][0].wait()
        gadd_h[ppw - 1][1].wait()
        issue_scat(ppw - 1)
        scat_h[ppw - 2].wait()
        scat_h[ppw - 1].wait()

    return sc_kernel


def kernel(x, pos_table):
    b, l, d = x.shape
    lw = l // WIDE
    x2 = x.reshape(b * lw, WIDE, d)
    pos2 = jnp.tile(pos_table.reshape(lw, WIDE, d), (PAIR, 1, 1))
    rows = jnp.arange(b * lw, dtype=jnp.int32).reshape(b // PAIR, PAIR * lw)
    idx_a = rows[:, :128]
    idx_b = rows[:, 128:]
    out = _make_sc_kernel(b, lw, d)(x2, pos2, idx_a, idx_b)
    return out.reshape(b, l, d)
